# baseline (speedup 1.0000x reference)
import functools

import jax
import jax.numpy as jnp
from jax import lax
from jax.experimental import pallas as pl
from jax.experimental.pallas import tpu as pltpu
from jax.experimental.pallas import tpu_sc as plsc

K_TOP = 64


def _encode_kernel(x_ref, w_ref, benc_ref, bdec_ref, z_ref):
    xc = x_ref[...] - bdec_ref[...]
    acc = jnp.dot(xc, w_ref[...], preferred_element_type=jnp.float32)
    z_ref[...] = jnp.maximum(acc + benc_ref[...], 0.0)


def _scalarize(v):
    if getattr(v, "ndim", 0) == 0:
        return v
    return jnp.min(v)


_BISECT_ITERS = 28


def _thresh_kernel(z_ref, thr_ref, *, k):
    z = z_ref[...]
    row_max = jnp.max(z, axis=1, keepdims=True)
    lo = jnp.zeros_like(row_max)
    hi = row_max * 1.000001 + 1e-30

    def body(_, carry):
        lo, hi = carry
        mid = 0.5 * (lo + hi)
        cnt = jnp.sum((z >= mid).astype(jnp.float32), axis=1, keepdims=True)
        pred = cnt >= k
        return jnp.where(pred, mid, lo), jnp.where(pred, hi, mid)

    lo, hi = jax.lax.fori_loop(0, _BISECT_ITERS, body, (lo, hi))
    thr_ref[...] = jnp.broadcast_to(lo, thr_ref.shape)


def _make_sc_threshold(n_rows, d_sae, k, row_lo):
    info = plsc.get_sparse_core_info()
    nw = info.num_cores * info.num_subcores
    rows = n_rows // nw
    nvec = d_sae // 16
    mesh = plsc.VectorSubcoreMesh(core_axis_name="c", subcore_axis_name="s")

    @functools.partial(
        pl.kernel,
        mesh=mesh,
        out_type=jax.ShapeDtypeStruct((n_rows,), jnp.int32),
        scratch_types=[
            pltpu.VMEM((d_sae,), jnp.float32),
            pltpu.VMEM((d_sae,), jnp.float32),
            pltpu.VMEM((2048,), jnp.float32),
            pltpu.VMEM((rows,), jnp.int32),
            pltpu.SemaphoreType.DMA,
            pltpu.SemaphoreType.DMA,
        ],
        compiler_params=pltpu.CompilerParams(needs_layout_passes=False),
    )
    def sc_thr(z_hbm, thr_hbm, row0_v, row1_v, hist_v, thr_v,
               sem0, sem1):
        wid = lax.axis_index("s") * info.num_cores + lax.axis_index("c")
        obase = wid * rows
        base = row_lo + obase
        ones16 = jnp.ones((16,), jnp.float32)
        zeros16 = jnp.zeros((16,), jnp.float32)
        lane0 = jnp.arange(16, dtype=jnp.int32) == 0

        def pick(counts_vec, needed):
            r = lax.rev(counts_vec, (0,))
            c = plsc.cumsum(r)
            mask = c >= needed
            l0 = _scalarize(plsc.all_reduce_ffs(mask))
            above = jnp.max(jnp.where(mask, 0.0, c))
            sel = jnp.min(jnp.where(mask, c, jnp.float32(3e7)))
            return 15 - l0, needed - above, sel - above

        def level(nitems, vec_of, needed):
            star = jnp.int32(0)
            before = jnp.float32(0)
            carry = jnp.float32(0)
            for g in range(nitems - 1, -1, -1):
                s = jnp.sum(vec_of(g))
                found = jnp.logical_and(carry < needed, carry + s >= needed)
                star = jnp.where(found, jnp.int32(g), star)
                before = jnp.where(found, carry, before)
                carry = carry + s
            return star, needed - before

        def sup16(base_vec):
            v = [hist_v[pl.ds((base_vec + t) * 16, 16)] for t in range(16)]
            while len(v) > 1:
                v = [v[i] + v[i + 1] for i in range(0, len(v), 2)]
            return v[0]

        def scan_h(nvecs, needed):
            g, needed = level(nvecs // 16, lambda g: sup16(g * 16), needed)
            kx, needed = level(
                16, lambda kk: hist_v[pl.ds((g * 16 + kk) * 16, 16)],
                needed)
            vec_ix = g * 16 + kx
            l, needed, cnt = pick(hist_v[pl.ds(vec_ix * 16, 16)], needed)
            return vec_ix * 16 + l, needed, cnt

        def zero_hists():
            for i in range(128):
                hist_v[pl.ds(i * 16, 16)] = zeros16

        def radix_row(row_ref):
            zero_hists()

            @plsc.parallel_loop(0, nvec, unroll=8)
            def _pa(j):
                bits = lax.bitcast_convert_type(
                    row_ref[pl.ds(j * 16, 16)], jnp.int32)
                plsc.addupdate_scatter(
                    hist_v, [lax.shift_right_logical(bits, 20)], ones16)

            b1, m1, _ = scan_h(128, jnp.float32(k))

            zero_hists()

            @plsc.parallel_loop(0, nvec, unroll=8)
            def _pb(j):
                bits = lax.bitcast_convert_type(
                    row_ref[pl.ds(j * 16, 16)], jnp.int32)
                msk = lax.shift_right_logical(bits, 20) == b1
                b2v = lax.shift_right_logical(bits, 9) & 0x7FF
                plsc.addupdate_scatter(hist_v, [b2v], ones16, mask=msk)

            b2, m2, _ = scan_h(128, m1)
            top22 = (b1 << 11) | b2

            for i in range(32):
                hist_v[pl.ds(i * 16, 16)] = zeros16

            @plsc.parallel_loop(0, nvec, unroll=8)
            def _pc(j):
                bits = lax.bitcast_convert_type(
                    row_ref[pl.ds(j * 16, 16)], jnp.int32)
                msk = lax.shift_right_logical(bits, 9) == top22
                plsc.addupdate_scatter(
                    hist_v, [bits & 0x1FF], ones16, mask=msk)

            b3, _, _ = scan_h(32, m2)
            return (top22 << 9) | b3

        def put_thr(r, t_bits):
            plsc.store_scatter(
                thr_v, [jnp.full((16,), r, jnp.int32)],
                jnp.full((16,), t_bits, jnp.int32), mask=lane0)

        pltpu.async_copy(z_hbm.at[base], row0_v, sem0)

        def pair_body(p, _):
            r0 = base + 2 * p
            pltpu.async_copy(z_hbm.at[r0 + 1], row1_v, sem1)
            pltpu.make_async_copy(z_hbm.at[r0], row0_v, sem0).wait()
            put_thr(2 * p, radix_row(row0_v))

            @pl.when(2 * p + 2 < rows)
            def _prefetch():
                pltpu.async_copy(z_hbm.at[r0 + 2], row0_v, sem0)

            pltpu.make_async_copy(z_hbm.at[r0 + 1], row1_v, sem1).wait()
            put_thr(2 * p + 1, radix_row(row1_v))
            return 0

        lax.fori_loop(0, rows // 2, pair_body, 0)
        pltpu.sync_copy(thr_v, thr_hbm.at[pl.ds(obase, rows)])

    return sc_thr


def _decode_kernel(z_ref, w_ref, bdec_ref, thr_ref, xrec_ref, zsp_ref):
    zs = jnp.where(z_ref[...] >= thr_ref[:, :1], z_ref[...], 0.0)
    zsp_ref[...] = zs
    j = pl.program_id(1)

    @pl.when(j == 0)
    def _init():
        xrec_ref[...] = jnp.broadcast_to(bdec_ref[...], xrec_ref.shape)

    xrec_ref[...] += jnp.dot(zs, w_ref[...],
                             preferred_element_type=jnp.float32)


def kernel(x, W_enc, b_enc, W_dec, b_dec):
    n_tok, d_in = x.shape
    d_sae = W_enc.shape[1]
    f32 = jnp.float32

    b_enc2 = b_enc.reshape(1, d_sae)
    b_dec2 = b_dec.reshape(1, d_in)

    tb = min(1024, n_tok)
    sb = min(1024, d_sae)
    nt, ns = n_tok // tb, d_sae // sb
    z = pl.pallas_call(
        _encode_kernel,
        grid=(ns, nt),
        in_specs=[
            pl.BlockSpec((tb, d_in), lambda j, i: (i, 0)),
            pl.BlockSpec((d_in, sb), lambda j, i: (0, j)),
            pl.BlockSpec((1, sb), lambda j, i: (0, j)),
            pl.BlockSpec((1, d_in), lambda j, i: (0, 0)),
        ],
        out_specs=pl.BlockSpec((tb, sb), lambda j, i: (i, j)),
        out_shape=jax.ShapeDtypeStruct((n_tok, d_sae), f32),
        compiler_params=pltpu.CompilerParams(
            dimension_semantics=("arbitrary", "arbitrary"),
        ),
    )(x, W_enc, b_enc2, b_dec2)

    n_sc = (n_tok // 2) // 512 * 512
    n_tc = n_tok - n_sc
    thr_bits = _make_sc_threshold(n_sc, d_sae, K_TOP, n_tc)(z)
    mb = min(128, n_tc)
    thr_tc = pl.pallas_call(
        functools.partial(_thresh_kernel, k=K_TOP),
        grid=(n_tc // mb,),
        in_specs=[pl.BlockSpec((mb, d_sae), lambda i: (i, 0))],
        out_specs=pl.BlockSpec((mb, 128), lambda i: (i, 0)),
        out_shape=jax.ShapeDtypeStruct((n_tc, 128), f32),
    )(z)
    thr_sc = jnp.broadcast_to(
        jax.lax.bitcast_convert_type(thr_bits, f32)[:, None], (n_sc, 128))
    thr = jnp.concatenate([thr_tc, thr_sc], axis=0)

    tb2 = min(1024, n_tok)
    kb2 = min(1024, d_sae)
    x_rec, z_sparse = pl.pallas_call(
        _decode_kernel,
        grid=(n_tok // tb2, d_sae // kb2),
        in_specs=[
            pl.BlockSpec((tb2, kb2), lambda i, j: (i, j)),
            pl.BlockSpec((kb2, d_in), lambda i, j: (j, 0)),
            pl.BlockSpec((1, d_in), lambda i, j: (0, 0)),
            pl.BlockSpec((tb2, 128), lambda i, j: (i, 0)),
        ],
        out_specs=[
            pl.BlockSpec((tb2, d_in), lambda i, j: (i, 0)),
            pl.BlockSpec((tb2, kb2), lambda i, j: (i, j)),
        ],
        out_shape=[
            jax.ShapeDtypeStruct((n_tok, d_in), f32),
            jax.ShapeDtypeStruct((n_tok, d_sae), f32),
        ],
        compiler_params=pltpu.CompilerParams(
            dimension_semantics=("parallel", "arbitrary"),
        ),
    )(z, W_dec, b_dec2, thr)

    return (x_rec, z_sparse)


# speedup vs baseline: 1.1476x; 1.1476x over previous
import functools

import jax
import jax.numpy as jnp
from jax import lax
from jax.experimental import pallas as pl
from jax.experimental.pallas import tpu as pltpu
from jax.experimental.pallas import tpu_sc as plsc

K_TOP = 64


def _encode_kernel(x_ref, w_ref, benc_ref, bdec_ref, z_ref):
    xc = x_ref[...] - bdec_ref[...]
    acc = jnp.dot(xc, w_ref[...], preferred_element_type=jnp.float32)
    z_ref[...] = jnp.maximum(acc + benc_ref[...], 0.0)


def _scalarize(v):
    if getattr(v, "ndim", 0) == 0:
        return v
    return jnp.min(v)


_BISECT_ITERS = 28


def _thresh_kernel(z_ref, thr_ref, *, k):
    z = z_ref[...]
    row_max = jnp.max(z, axis=1, keepdims=True)
    lo = jnp.zeros_like(row_max)
    hi = row_max * 1.000001 + 1e-30

    def body(_, carry):
        lo, hi = carry
        mid = 0.5 * (lo + hi)
        cnt = jnp.sum((z >= mid).astype(jnp.float32), axis=1, keepdims=True)
        pred = cnt >= k
        return jnp.where(pred, mid, lo), jnp.where(pred, hi, mid)

    lo, hi = jax.lax.fori_loop(0, _BISECT_ITERS, body, (lo, hi))
    thr_ref[...] = jnp.broadcast_to(lo, thr_ref.shape)


def _make_sc_threshold(n_rows, d_sae, k, row_lo):
    info = plsc.get_sparse_core_info()
    nw = info.num_cores * info.num_subcores
    rows = n_rows // nw
    nvec = d_sae // 16
    mesh = plsc.VectorSubcoreMesh(core_axis_name="c", subcore_axis_name="s")

    @functools.partial(
        pl.kernel,
        mesh=mesh,
        out_type=jax.ShapeDtypeStruct((n_rows,), jnp.int32),
        scratch_types=[
            pltpu.VMEM((d_sae,), jnp.float32),
            pltpu.VMEM((d_sae,), jnp.float32),
            pltpu.VMEM((2048,), jnp.float32),
            pltpu.VMEM((rows,), jnp.int32),
            pltpu.SemaphoreType.DMA,
            pltpu.SemaphoreType.DMA,
        ],
        compiler_params=pltpu.CompilerParams(needs_layout_passes=False),
    )
    def sc_thr(z_hbm, thr_hbm, row0_v, row1_v, hist_v, thr_v,
               sem0, sem1):
        wid = lax.axis_index("s") * info.num_cores + lax.axis_index("c")
        obase = wid * rows
        base = row_lo + obase
        ones16 = jnp.ones((16,), jnp.float32)
        zeros16 = jnp.zeros((16,), jnp.float32)
        lane0 = jnp.arange(16, dtype=jnp.int32) == 0

        def pick(counts_vec, needed):
            r = lax.rev(counts_vec, (0,))
            c = plsc.cumsum(r)
            mask = c >= needed
            l0 = _scalarize(plsc.all_reduce_ffs(mask))
            above = jnp.max(jnp.where(mask, 0.0, c))
            sel = jnp.min(jnp.where(mask, c, jnp.float32(3e7)))
            return 15 - l0, needed - above, sel - above

        def level(nitems, vec_of, needed):
            star = jnp.int32(0)
            before = jnp.float32(0)
            carry = jnp.float32(0)
            for g in range(nitems - 1, -1, -1):
                s = jnp.sum(vec_of(g))
                found = jnp.logical_and(carry < needed, carry + s >= needed)
                star = jnp.where(found, jnp.int32(g), star)
                before = jnp.where(found, carry, before)
                carry = carry + s
            return star, needed - before

        def sup16(base_vec):
            v = [hist_v[pl.ds((base_vec + t) * 16, 16)] for t in range(16)]
            while len(v) > 1:
                v = [v[i] + v[i + 1] for i in range(0, len(v), 2)]
            return v[0]

        def scan_h(nvecs, needed):
            g, needed = level(nvecs // 16, lambda g: sup16(g * 16), needed)
            kx, needed = level(
                16, lambda kk: hist_v[pl.ds((g * 16 + kk) * 16, 16)],
                needed)
            vec_ix = g * 16 + kx
            l, needed, cnt = pick(hist_v[pl.ds(vec_ix * 16, 16)], needed)
            return vec_ix * 16 + l, needed, cnt

        def zero_hists():
            for i in range(128):
                hist_v[pl.ds(i * 16, 16)] = zeros16

        def radix_row(row_ref):
            zero_hists()

            @plsc.parallel_loop(0, nvec, unroll=8)
            def _pa(j):
                bits = lax.bitcast_convert_type(
                    row_ref[pl.ds(j * 16, 16)], jnp.int32)
                plsc.addupdate_scatter(
                    hist_v, [lax.shift_right_logical(bits, 20)], ones16)

            b1, m1, _ = scan_h(128, jnp.float32(k))

            zero_hists()

            @plsc.parallel_loop(0, nvec, unroll=8)
            def _pb(j):
                bits = lax.bitcast_convert_type(
                    row_ref[pl.ds(j * 16, 16)], jnp.int32)
                msk = lax.shift_right_logical(bits, 20) == b1
                b2v = lax.shift_right_logical(bits, 9) & 0x7FF
                plsc.addupdate_scatter(hist_v, [b2v], ones16, mask=msk)

            b2, m2, _ = scan_h(128, m1)
            top22 = (b1 << 11) | b2

            for i in range(32):
                hist_v[pl.ds(i * 16, 16)] = zeros16

            @plsc.parallel_loop(0, nvec, unroll=8)
            def _pc(j):
                bits = lax.bitcast_convert_type(
                    row_ref[pl.ds(j * 16, 16)], jnp.int32)
                msk = lax.shift_right_logical(bits, 9) == top22
                plsc.addupdate_scatter(
                    hist_v, [bits & 0x1FF], ones16, mask=msk)

            b3, _, _ = scan_h(32, m2)
            return (top22 << 9) | b3

        def put_thr(r, t_bits):
            plsc.store_scatter(
                thr_v, [jnp.full((16,), r, jnp.int32)],
                jnp.full((16,), t_bits, jnp.int32), mask=lane0)

        pltpu.async_copy(z_hbm.at[base], row0_v, sem0)

        def pair_body(p, _):
            r0 = base + 2 * p
            pltpu.async_copy(z_hbm.at[r0 + 1], row1_v, sem1)
            pltpu.make_async_copy(z_hbm.at[r0], row0_v, sem0).wait()
            put_thr(2 * p, radix_row(row0_v))

            @pl.when(2 * p + 2 < rows)
            def _prefetch():
                pltpu.async_copy(z_hbm.at[r0 + 2], row0_v, sem0)

            pltpu.make_async_copy(z_hbm.at[r0 + 1], row1_v, sem1).wait()
            put_thr(2 * p + 1, radix_row(row1_v))
            return 0

        lax.fori_loop(0, rows // 2, pair_body, 0)
        pltpu.sync_copy(thr_v, thr_hbm.at[pl.ds(obase, rows)])

    return sc_thr


def _decode_kernel(z_ref, w_ref, bdec_ref, thr_ref, xrec_ref, zsp_ref):
    zs = jnp.where(z_ref[...] >= thr_ref[:, :1], z_ref[...], 0.0)
    zsp_ref[...] = zs
    j = pl.program_id(1)

    @pl.when(j == 0)
    def _init():
        xrec_ref[...] = jnp.broadcast_to(bdec_ref[...], xrec_ref.shape)

    xrec_ref[...] += jnp.dot(zs, w_ref[...],
                             preferred_element_type=jnp.float32)


def kernel(x, W_enc, b_enc, W_dec, b_dec):
    n_tok, d_in = x.shape
    d_sae = W_enc.shape[1]
    f32 = jnp.float32

    b_enc2 = b_enc.reshape(1, d_sae)
    b_dec2 = b_dec.reshape(1, d_in)

    tb = min(1024, n_tok)
    sb = min(1024, d_sae)
    nt, ns = n_tok // tb, d_sae // sb
    z = pl.pallas_call(
        _encode_kernel,
        grid=(ns, nt),
        in_specs=[
            pl.BlockSpec((tb, d_in), lambda j, i: (i, 0)),
            pl.BlockSpec((d_in, sb), lambda j, i: (0, j)),
            pl.BlockSpec((1, sb), lambda j, i: (0, j)),
            pl.BlockSpec((1, d_in), lambda j, i: (0, 0)),
        ],
        out_specs=pl.BlockSpec((tb, sb), lambda j, i: (i, j)),
        out_shape=jax.ShapeDtypeStruct((n_tok, d_sae), f32),
        compiler_params=pltpu.CompilerParams(
            dimension_semantics=("arbitrary", "arbitrary"),
        ),
    )(x, W_enc, b_enc2, b_dec2)

    n_sc = (3 * n_tok // 8) // 512 * 512
    n_tc = n_tok - n_sc
    thr_bits = _make_sc_threshold(n_sc, d_sae, K_TOP, n_tc)(z)
    mb = min(128, n_tc)
    thr_tc = pl.pallas_call(
        functools.partial(_thresh_kernel, k=K_TOP),
        grid=(n_tc // mb,),
        in_specs=[pl.BlockSpec((mb, d_sae), lambda i: (i, 0))],
        out_specs=pl.BlockSpec((mb, 128), lambda i: (i, 0)),
        out_shape=jax.ShapeDtypeStruct((n_tc, 128), f32),
    )(z)
    thr_sc = jnp.broadcast_to(
        jax.lax.bitcast_convert_type(thr_bits, f32)[:, None], (n_sc, 128))
    thr = jnp.concatenate([thr_tc, thr_sc], axis=0)

    tb2 = min(1024, n_tok)
    kb2 = min(1024, d_sae)
    x_rec, z_sparse = pl.pallas_call(
        _decode_kernel,
        grid=(n_tok // tb2, d_sae // kb2),
        in_specs=[
            pl.BlockSpec((tb2, kb2), lambda i, j: (i, j)),
            pl.BlockSpec((kb2, d_in), lambda i, j: (j, 0)),
            pl.BlockSpec((1, d_in), lambda i, j: (0, 0)),
            pl.BlockSpec((tb2, 128), lambda i, j: (i, 0)),
        ],
        out_specs=[
            pl.BlockSpec((tb2, d_in), lambda i, j: (i, 0)),
            pl.BlockSpec((tb2, kb2), lambda i, j: (i, j)),
        ],
        out_shape=[
            jax.ShapeDtypeStruct((n_tok, d_in), f32),
            jax.ShapeDtypeStruct((n_tok, d_sae), f32),
        ],
        compiler_params=pltpu.CompilerParams(
            dimension_semantics=("parallel", "arbitrary"),
        ),
    )(z, W_dec, b_dec2, thr)

    return (x_rec, z_sparse)
